# Initial kernel scaffold; baseline (speedup 1.0000x reference)
import jax
import jax.numpy as jnp
from jax.experimental import pallas as pl


def kernel(x):
    raise NotImplementedError("write your pallas kernel here")


# speedup vs baseline: 2.9723x; 2.9723x over previous
import functools

import jax
import jax.numpy as jnp
from jax import lax
from jax.experimental import pallas as pl
from jax.experimental.pallas import tpu as pltpu
from jax.experimental.pallas import tpu_sc as plsc

_B, _S, _D = 4, 8192, 1024
_ROWS_OUT = _B * (_S // 2)
_NW = 32
_ROWS_PER_W = _ROWS_OUT // _NW
_CHUNK_ROWS = 16
_CHUNKS = _ROWS_PER_W // _CHUNK_ROWS
_IN_WORDS = _CHUNK_ROWS * 2 * _D
_OUT_WORDS = _CHUNK_ROWS * _D
_LANES = 16


def _avg_pool_sc(xf):
    mesh = plsc.VectorSubcoreMesh(core_axis_name="c", subcore_axis_name="s")

    @functools.partial(
        pl.kernel,
        mesh=mesh,
        out_type=jax.ShapeDtypeStruct((_ROWS_OUT * _D,), jnp.float32),
        scratch_types=[
            pltpu.VMEM((_IN_WORDS,), jnp.float32),
            pltpu.VMEM((_OUT_WORDS,), jnp.float32),
        ],
    )
    def k(x_hbm, o_hbm, in_v, out_v):
        wid = lax.axis_index("s") * 2 + lax.axis_index("c")

        def chunk_body(g, carry):
            in_off = wid * (_ROWS_PER_W * 2 * _D) + g * _IN_WORDS
            out_off = wid * (_ROWS_PER_W * _D) + g * _OUT_WORDS
            pltpu.sync_copy(x_hbm.at[pl.ds(in_off, _IN_WORDS)], in_v)

            def row_body(r, c2):
                bi = r * (2 * _D)
                bo = r * _D
                for j in range(_D // _LANES):
                    a = in_v[pl.ds(bi + j * _LANES, _LANES)]
                    b = in_v[pl.ds(bi + _D + j * _LANES, _LANES)]
                    out_v[pl.ds(bo + j * _LANES, _LANES)] = (a + b) * 0.5
                return c2

            lax.fori_loop(0, _CHUNK_ROWS, row_body, 0)
            pltpu.sync_copy(out_v, o_hbm.at[pl.ds(out_off, _OUT_WORDS)])
            return carry

        lax.fori_loop(0, _CHUNKS, chunk_body, 0)

    return k(xf)


def kernel(x):
    xf = x.reshape(-1)
    of = _avg_pool_sc(xf)
    return of.reshape(_B, _S // 2, _D)


# speedup vs baseline: 4.6140x; 1.5523x over previous
import functools

import jax
import jax.numpy as jnp
from jax import lax
from jax.experimental import pallas as pl
from jax.experimental.pallas import tpu as pltpu
from jax.experimental.pallas import tpu_sc as plsc

_B, _S, _D = 4, 8192, 1024
_ROWS_OUT = _B * (_S // 2)
_NW = 32
_ROWS_PER_W = _ROWS_OUT // _NW
_CHUNK_ROWS = 16
_CHUNKS = _ROWS_PER_W // _CHUNK_ROWS
_IN_WORDS = _CHUNK_ROWS * 2 * _D
_OUT_WORDS = _CHUNK_ROWS * _D
_LANES = 16


def _avg_pool_sc(xf):
    mesh = plsc.VectorSubcoreMesh(core_axis_name="c", subcore_axis_name="s")

    @functools.partial(
        pl.kernel,
        mesh=mesh,
        out_type=jax.ShapeDtypeStruct((_ROWS_OUT * _D,), jnp.float32),
        scratch_types=[
            pltpu.VMEM((_IN_WORDS,), jnp.float32),
            pltpu.VMEM((_OUT_WORDS,), jnp.float32),
        ],
    )
    def k(x_hbm, o_hbm, in_v, out_v):
        wid = lax.axis_index("s") * 2 + lax.axis_index("c")

        def chunk_body(g, carry):
            in_off = wid * (_ROWS_PER_W * 2 * _D) + g * _IN_WORDS
            out_off = wid * (_ROWS_PER_W * _D) + g * _OUT_WORDS
            pltpu.sync_copy(x_hbm.at[pl.ds(in_off, _IN_WORDS)], in_v)

            @plsc.parallel_loop(0, _OUT_WORDS // _LANES, unroll=8)
            def vec_body(j):
                row = j >> 6
                col = j & (_D // _LANES - 1)
                bi = row * (2 * _D) + col * _LANES
                a = in_v[pl.ds(bi, _LANES)]
                b = in_v[pl.ds(bi + _D, _LANES)]
                out_v[pl.ds(j * _LANES, _LANES)] = (a + b) * 0.5

            pltpu.sync_copy(out_v, o_hbm.at[pl.ds(out_off, _OUT_WORDS)])
            return carry

        lax.fori_loop(0, _CHUNKS, chunk_body, 0)

    return k(xf)


def kernel(x):
    xf = x.reshape(-1)
    of = _avg_pool_sc(xf)
    return of.reshape(_B, _S // 2, _D)


# speedup vs baseline: 5.5991x; 1.2135x over previous
import functools

import jax
import jax.numpy as jnp
from jax import lax
from jax.experimental import pallas as pl
from jax.experimental.pallas import tpu as pltpu
from jax.experimental.pallas import tpu_sc as plsc

_B, _S, _D = 4, 8192, 1024
_ROWS_OUT = _B * (_S // 2)
_NW = 32
_ROWS_PER_W = _ROWS_OUT // _NW
_CHUNK_ROWS = 16
_CHUNKS = _ROWS_PER_W // _CHUNK_ROWS
_IN_WORDS = _CHUNK_ROWS * 2 * _D
_OUT_WORDS = _CHUNK_ROWS * _D
_LANES = 16


def _avg_pool_sc(xf):
    mesh = plsc.VectorSubcoreMesh(core_axis_name="c", subcore_axis_name="s")

    @functools.partial(
        pl.kernel,
        mesh=mesh,
        out_type=jax.ShapeDtypeStruct((_ROWS_OUT * _D,), jnp.float32),
        scratch_types=[
            pltpu.VMEM((_IN_WORDS,), jnp.float32),
            pltpu.VMEM((_IN_WORDS,), jnp.float32),
            pltpu.VMEM((_OUT_WORDS,), jnp.float32),
            pltpu.VMEM((_OUT_WORDS,), jnp.float32),
            pltpu.SemaphoreType.DMA,
            pltpu.SemaphoreType.DMA,
            pltpu.SemaphoreType.DMA,
            pltpu.SemaphoreType.DMA,
        ],
    )
    def k(x_hbm, o_hbm, in_v0, in_v1, out_v0, out_v1, si0, si1, so0, so1):
        wid = lax.axis_index("s") * 2 + lax.axis_index("c")
        base_in = wid * (_ROWS_PER_W * 2 * _D)
        base_out = wid * (_ROWS_PER_W * _D)
        in_bufs, out_bufs = (in_v0, in_v1), (out_v0, out_v1)
        sin, sout = (si0, si1), (so0, so1)

        def in_copy(g, b):
            return pltpu.make_async_copy(
                x_hbm.at[pl.ds(base_in + g * _IN_WORDS, _IN_WORDS)],
                in_bufs[b], sin[b])

        def out_copy(g, b):
            return pltpu.make_async_copy(
                out_bufs[b],
                o_hbm.at[pl.ds(base_out + g * _OUT_WORDS, _OUT_WORDS)],
                sout[b])

        in_copy(0, 0).start()

        def outer(g2, carry):
            for b in range(2):
                g = g2 * 2 + b
                nb = 1 - b

                @pl.when(g + 1 < _CHUNKS)
                def _start_next():
                    in_copy(g + 1, nb).start()

                in_copy(g, b).wait()

                @pl.when(g >= 2)
                def _drain_prev():
                    out_copy(g - 2, b).wait()

                out_v = out_bufs[b]
                in_v = in_bufs[b]

                @plsc.parallel_loop(0, _OUT_WORDS // _LANES, unroll=8)
                def vec_body(j):
                    row = j >> 6
                    col = j & (_D // _LANES - 1)
                    bi = row * (2 * _D) + col * _LANES
                    a = in_v[pl.ds(bi, _LANES)]
                    bb = in_v[pl.ds(bi + _D, _LANES)]
                    out_v[pl.ds(j * _LANES, _LANES)] = (a + bb) * 0.5

                out_copy(g, b).start()
            return carry

        lax.fori_loop(0, _CHUNKS // 2, outer, 0)
        for b in range(2):
            out_copy(_CHUNKS - 2 + b, b).wait()

    return k(xf)


def kernel(x):
    xf = x.reshape(-1)
    of = _avg_pool_sc(xf)
    return of.reshape(_B, _S // 2, _D)


# speedup vs baseline: 15.6580x; 2.7965x over previous
import functools

import jax
import jax.numpy as jnp
from jax import lax
from jax.experimental import pallas as pl
from jax.experimental.pallas import tpu as pltpu
from jax.experimental.pallas import tpu_sc as plsc

_B, _S, _D = 4, 8192, 1024
_ROWS_OUT = _B * (_S // 2)
_NW = 32
_ROWS_PER_W = _ROWS_OUT // _NW
_CHUNK_ROWS = 16
_CHUNKS = _ROWS_PER_W // _CHUNK_ROWS
_LANES = 16


def _avg_pool_sc(x2):
    mesh = plsc.VectorSubcoreMesh(core_axis_name="c", subcore_axis_name="s")

    @functools.partial(
        pl.kernel,
        mesh=mesh,
        out_type=jax.ShapeDtypeStruct((_ROWS_OUT, _D), jnp.float32),
        scratch_types=[
            pltpu.VMEM((2 * _CHUNK_ROWS, _D), jnp.float32),
            pltpu.VMEM((2 * _CHUNK_ROWS, _D), jnp.float32),
            pltpu.VMEM((_CHUNK_ROWS, _D), jnp.float32),
            pltpu.VMEM((_CHUNK_ROWS, _D), jnp.float32),
            pltpu.SemaphoreType.DMA,
            pltpu.SemaphoreType.DMA,
            pltpu.SemaphoreType.DMA,
            pltpu.SemaphoreType.DMA,
        ],
    )
    def k(x_hbm, o_hbm, in_v0, in_v1, out_v0, out_v1, si0, si1, so0, so1):
        wid = lax.axis_index("s") * 2 + lax.axis_index("c")
        base_in = wid * (_ROWS_PER_W * 2)
        base_out = wid * _ROWS_PER_W
        in_bufs, out_bufs = (in_v0, in_v1), (out_v0, out_v1)
        sin, sout = (si0, si1), (so0, so1)

        def in_copy(g, b):
            return pltpu.make_async_copy(
                x_hbm.at[pl.ds(base_in + g * 2 * _CHUNK_ROWS, 2 * _CHUNK_ROWS)],
                in_bufs[b], sin[b])

        def out_copy(g, b):
            return pltpu.make_async_copy(
                out_bufs[b],
                o_hbm.at[pl.ds(base_out + g * _CHUNK_ROWS, _CHUNK_ROWS)],
                sout[b])

        in_copy(0, 0).start()

        def outer(g2, carry):
            for b in range(2):
                g = g2 * 2 + b
                nb = 1 - b

                @pl.when(g + 1 < _CHUNKS)
                def _start_next():
                    in_copy(g + 1, nb).start()

                in_copy(g, b).wait()

                @pl.when(g >= 2)
                def _drain_prev():
                    out_copy(g - 2, b).wait()

                out_v = out_bufs[b]
                in_v = in_bufs[b]

                @plsc.parallel_loop(0, _CHUNK_ROWS * (_D // _LANES), unroll=8)
                def vec_body(j):
                    row = j >> 6
                    col = (j & (_D // _LANES - 1)) * _LANES
                    a = in_v[2 * row, pl.ds(col, _LANES)]
                    bb = in_v[2 * row + 1, pl.ds(col, _LANES)]
                    out_v[row, pl.ds(col, _LANES)] = (a + bb) * 0.5

                out_copy(g, b).start()
            return carry

        lax.fori_loop(0, _CHUNKS // 2, outer, 0)
        for b in range(2):
            out_copy(_CHUNKS - 2 + b, b).wait()

    return k(x2)


def kernel(x):
    x2 = x.reshape(_ROWS_OUT * 2, _D)
    of = _avg_pool_sc(x2)
    return of.reshape(_B, _S // 2, _D)
